# Initial kernel scaffold; baseline (speedup 1.0000x reference)
import jax
import jax.numpy as jnp
from jax.experimental import pallas as pl


def kernel(image_features, codebooks):
    raise NotImplementedError("write your pallas kernel here")


# speedup vs baseline: 1.6768x; 1.6768x over previous
import functools

import jax
import jax.numpy as jnp
from jax import lax
from jax.experimental import pallas as pl
from jax.experimental.pallas import tpu as pltpu
from jax.experimental.pallas import tpu_sc as plsc

B, D, Q, K = 2048, 768, 8, 4096

TKP = 1024
NKP = K // TKP
TK = 512
NKT = K // TK

NC, NS = 2, 16
NW = NC * NS
BPW = B // NW


def _prep_body(cb_ref, cbn_ref, cbt_ref, ortho_ref, gram_acc):
    kp = pl.program_id(1)
    x = cb_ref[0]
    sq = jnp.sum(x * x, axis=1, keepdims=True)
    cbn = x / jnp.maximum(jnp.sqrt(sq), 1e-12)
    cbn_ref[0] = cbn
    cbn_bf = cbn.astype(jnp.bfloat16)
    cbt_ref[0] = cbn_bf.T
    g = jax.lax.dot_general(
        cbn_bf.T, cbn_bf,
        (((1,), (0,)), ((), ())), preferred_element_type=jnp.float32)

    @pl.when(kp == 0)
    def _():
        gram_acc[...] = g

    @pl.when(kp > 0)
    def _():
        gram_acc[...] = gram_acc[...] + g

    @pl.when(kp == NKP - 1)
    def _():
        ss = jnp.sum(gram_acc[...] * gram_acc[...])
        val = (ss - jnp.float32(K)) / jnp.float32(K * K)
        ortho_ref[...] = jnp.full((1, 8, 128), val, jnp.float32)


_prep_call = pl.pallas_call(
    _prep_body,
    grid=(Q, NKP),
    in_specs=[pl.BlockSpec((1, TKP, D), lambda q, k: (q, k, 0))],
    out_specs=[
        pl.BlockSpec((1, TKP, D), lambda q, k: (q, k, 0)),
        pl.BlockSpec((1, D, TKP), lambda q, k: (q, 0, k)),
        pl.BlockSpec((1, 8, 128), lambda q, k: (q, 0, 0)),
    ],
    out_shape=[
        jax.ShapeDtypeStruct((Q, K, D), jnp.float32),
        jax.ShapeDtypeStruct((Q, D, K), jnp.bfloat16),
        jax.ShapeDtypeStruct((Q, 8, 128), jnp.float32),
    ],
    scratch_shapes=[pltpu.VMEM((D, D), jnp.float32)],
)


def _stage_body(q, first, *refs):
    if first:
        (r_ref, ct_ref, rout_ref, idxraw_ref, idxflat_ref, commit_ref,
         rn_scr, rowsq_scr, bestv_scr, besti_scr) = refs
    else:
        (r_ref, qt_ref, ct_ref, rout_ref, idxraw_ref, idxflat_ref, commit_ref,
         rn_scr, rowsq_scr, bestv_scr, besti_scr) = refs
    kt = pl.program_id(0)

    @pl.when(kt == 0)
    def _():
        if first:
            rq = r_ref[...]
        else:
            rq = r_ref[...] - qt_ref[...]
        rout_ref[...] = rq
        rowsq = jnp.sum(rq * rq, axis=1, keepdims=True)
        rowsq_scr[...] = rowsq
        rn = rq / jnp.maximum(jnp.sqrt(rowsq), 1e-12)
        rn_scr[...] = rn.astype(jnp.bfloat16)
        bestv_scr[...] = jnp.full((B, 1), -jnp.inf, jnp.float32)
        besti_scr[...] = jnp.zeros((B, 1), jnp.int32)

    ct = ct_ref[0]
    s = jax.lax.dot_general(rn_scr[...], ct, (((1,), (0,)), ((), ())),
                            preferred_element_type=jnp.float32)
    m = jnp.max(s, axis=1, keepdims=True)
    iota = lax.broadcasted_iota(jnp.int32, (B, TK), 1)
    li = jnp.min(jnp.where(s == m, iota, jnp.int32(2**30)),
                 axis=1, keepdims=True)
    upd = m > bestv_scr[...]
    besti_scr[...] = jnp.where(upd, li + kt * TK, besti_scr[...])
    bestv_scr[...] = jnp.where(upd, m, bestv_scr[...])

    @pl.when(kt == NKT - 1)
    def _():
        rowsq = rowsq_scr[...]
        commit = jnp.mean(rowsq - 2.0 * bestv_scr[...] * jnp.sqrt(rowsq) + 1.0)
        commit_ref[...] = jnp.full((1, 128), commit, jnp.float32)
        idxraw_ref[...] = besti_scr[...]
        idxflat_ref[...] = besti_scr[...] + jnp.int32(q * K)


def _make_stage_call(q):
    first = q == 0
    full = pl.BlockSpec((B, D), lambda k: (0, 0))
    in_specs = [full] + ([] if first else [full]) + [
        pl.BlockSpec((1, D, TK), lambda k, _q=q: (_q, 0, k)),
    ]
    return pl.pallas_call(
        functools.partial(_stage_body, q, first),
        grid=(NKT,),
        in_specs=in_specs,
        out_specs=[
            full,
            pl.BlockSpec((B, 1), lambda k: (0, 0)),
            pl.BlockSpec((B, 1), lambda k: (0, 0)),
            pl.BlockSpec((1, 128), lambda k: (0, 0)),
        ],
        out_shape=[
            jax.ShapeDtypeStruct((B, D), jnp.float32),
            jax.ShapeDtypeStruct((B, 1), jnp.int32),
            jax.ShapeDtypeStruct((B, 1), jnp.int32),
            jax.ShapeDtypeStruct((1, 128), jnp.float32),
        ],
        scratch_shapes=[
            pltpu.VMEM((B, D), jnp.bfloat16),
            pltpu.VMEM((B, 1), jnp.float32),
            pltpu.VMEM((B, 1), jnp.float32),
            pltpu.VMEM((B, 1), jnp.int32),
        ],
    )


_stage_calls = [_make_stage_call(q) for q in range(Q)]


@functools.cache
def _get_sc_gather():
    mesh = plsc.VectorSubcoreMesh(
        core_axis_name="c", subcore_axis_name="s",
        num_cores=NC, num_subcores=NS)

    @functools.partial(
        pl.kernel,
        out_type=jax.ShapeDtypeStruct((B, D), jnp.float32),
        mesh=mesh,
        scratch_types=[
            pltpu.VMEM((BPW,), jnp.int32),
            pltpu.VMEM((BPW, D), jnp.float32),
            pltpu.SemaphoreType.DMA,
        ],
    )
    def _sc_gather(table_hbm, idx_hbm, out_hbm, idx_v, rows_v, sem):
        wid = lax.axis_index("s") * NC + lax.axis_index("c")
        base = wid * BPW
        pltpu.sync_copy(idx_hbm.at[pl.ds(base, BPW)], idx_v)
        pltpu.async_copy(table_hbm.at[idx_v], rows_v, sem).wait()
        pltpu.sync_copy(rows_v, out_hbm.at[pl.ds(base, BPW)])

    return _sc_gather


def _combine_body(x_ref, r_ref, qt_ref, out_ref):
    out_ref[...] = x_ref[...] - r_ref[...] + qt_ref[...]


_combine_call = pl.pallas_call(
    _combine_body,
    grid=(8,),
    in_specs=[pl.BlockSpec((B // 8, D), lambda i: (i, 0))] * 3,
    out_specs=pl.BlockSpec((B // 8, D), lambda i: (i, 0)),
    out_shape=jax.ShapeDtypeStruct((B, D), jnp.float32),
)


def kernel(image_features, codebooks):
    x = image_features
    cbn, cbt, ortho = _prep_call(codebooks)
    cbn_flat = cbn.reshape(Q * K, D)

    r = x
    quant = None
    idx_cols = []
    commits = []
    for q in range(Q):
        args = (r, cbt) if q == 0 else (r, quant, cbt)
        r, idxraw, idxflat, commit = _stage_calls[q](*args)
        quant = _get_sc_gather()(cbn_flat, idxflat.reshape(B))
        idx_cols.append(idxraw)
        commits.append(commit[0, 0])

    quantized = _combine_call(x, r, quant)
    indices = jnp.concatenate(idx_cols, axis=1)
    losses = jnp.stack(commits) + 10.0 * ortho[:, 0, 0]
    return quantized, indices, losses
